# baseline (speedup 1.0000x reference)
import functools

import jax
import jax.numpy as jnp
import numpy as np
from jax.experimental import pallas as pl
from jax.experimental.pallas import tpu as pltpu

IN_CH = 768
K = 9
GROUPS = 32
H = W = 16
N = H * W
B = 16

_F32_MIN = -3.0e38


def _sincos_1d(embed_dim, pos):
    omega = np.arange(embed_dim // 2, dtype=np.float64)
    omega = omega / (embed_dim / 2.0)
    omega = 1.0 / (10000.0 ** omega)
    out = np.einsum('m,d->md', pos.reshape(-1).astype(np.float64), omega)
    return np.concatenate([np.sin(out), np.cos(out)], axis=1)


def _relative_pos_np(embed_dim, grid_size):
    gh = np.arange(grid_size, dtype=np.float32)
    gw = np.arange(grid_size, dtype=np.float32)
    grid = np.stack(np.meshgrid(gw, gh), axis=0)
    emb_h = _sincos_1d(embed_dim // 2, grid[0])
    emb_w = _sincos_1d(embed_dim // 2, grid[1])
    pe = np.concatenate([emb_h, emb_w], axis=1)
    rp = 2.0 * (pe @ pe.T) / pe.shape[1]
    return rp.astype(np.float32)


def _group_map(channels, groups):
    m = np.zeros((channels, groups), dtype=np.float32)
    per = channels // groups
    for g in range(groups):
        m[g * per:(g + 1) * per, g] = 1.0
    return m


def _dot(a, b, precision=jax.lax.Precision.HIGHEST):
    return jax.lax.dot_general(a, b, (((1,), (0,)), ((), ())),
                               preferred_element_type=jnp.float32,
                               precision=precision)


def _group_norm(y, gmap, gmapT, gg, gb, nelem):
    s = jnp.sum(y, axis=0, keepdims=True)
    ss = jnp.sum(y * y, axis=0, keepdims=True)
    gs = _dot(s, gmap)
    gss = _dot(ss, gmap)
    mu = gs / nelem
    var = gss / nelem - mu * mu
    rstd = jax.lax.rsqrt(var + 1e-5)
    mu_c = _dot(mu, gmapT)
    rstd_c = _dot(rstd, gmapT)
    a = rstd_c * gg
    b = gb - mu_c * a
    return y * a + b


def _erf(z):
    return jax.lax.erf(z)


def _body(x_ref, fc1_wT_ref, fc1_b_ref, fc1_gg_ref, fc1_gb_ref,
          gw1_ref, gw2_ref, g_b_ref, g_gg_ref, g_gb_ref,
          fc2_wT_ref, fc2_b_ref, fc2_gg_ref, fc2_gb_ref,
          rel_pos_ref, gmapC_ref, gmapCT_ref, gmap2C_ref, gmap2CT_ref,
          out_ref,
          w1_v, g1_v, g2_v, f2_v, rp_v, s1, s2, s3, s4, s5):
    @pl.when(pl.program_id(0) == 0)
    def _stage():
        copies = [
            pltpu.make_async_copy(fc1_wT_ref, w1_v, s1),
            pltpu.make_async_copy(gw1_ref, g1_v, s2),
            pltpu.make_async_copy(gw2_ref, g2_v, s3),
            pltpu.make_async_copy(fc2_wT_ref, f2_v, s4),
            pltpu.make_async_copy(rel_pos_ref, rp_v, s5),
        ]
        for c in copies:
            c.start()
        for c in copies:
            c.wait()

    x = x_ref[0]
    y = _dot(x, w1_v[...], precision=None) + fc1_b_ref[...]
    xf = _group_norm(y, gmapC_ref[...], gmapCT_ref[...],
                     fc1_gg_ref[...], fc1_gb_ref[...],
                     float((IN_CH // GROUPS) * N))

    nrm = jnp.sqrt(jnp.sum(xf * xf, axis=1, keepdims=True))
    xn = xf / jnp.maximum(nrm, 1e-12)
    inner = jax.lax.dot_general(xn, xn, (((1,), (1,)), ((), ())),
                                preferred_element_type=jnp.float32,
                                precision=None)
    sq = jnp.sum(xn * xn, axis=1, keepdims=True)
    dist = 2.0 * inner - sq - jnp.transpose(sq) + rp_v[...]

    xf_bf = xf.astype(jnp.bfloat16)
    cols = jax.lax.broadcasted_iota(jnp.int32, (N, N), 1)
    d = dist
    acc = jnp.full((N, IN_CH), _F32_MIN, dtype=jnp.float32)
    for _ in range(K):
        m = jnp.max(d, axis=1, keepdims=True)
        amin = jnp.min(jnp.where(d == m, cols, N), axis=1, keepdims=True)
        onehot = (cols == amin).astype(jnp.bfloat16)
        acc = jnp.maximum(acc, _dot(onehot, xf_bf, precision=None))
        d = jnp.where(cols == amin, _F32_MIN, d)
    x_j = acc - xf

    t = (_dot(xf_bf, g1_v[...], precision=None)
         + _dot(x_j.astype(jnp.bfloat16), g2_v[...], precision=None)
         + g_b_ref[...])
    t = _group_norm(t, gmap2C_ref[...], gmap2CT_ref[...],
                    g_gg_ref[...], g_gb_ref[...],
                    float((2 * IN_CH // GROUPS) * N))
    u = 0.5 * t * (1.0 + _erf(t * np.float32(1.0 / np.sqrt(2.0))))

    z = _dot(u.astype(jnp.bfloat16), f2_v[...], precision=None) + fc2_b_ref[...]
    z = _group_norm(z, gmapC_ref[...], gmapCT_ref[...],
                    fc2_gg_ref[...], fc2_gb_ref[...],
                    float((IN_CH // GROUPS) * N))
    out_ref[0] = z + x


@functools.partial(jax.jit, static_argnames=())
def _run(xf_in, fc1_wT, fc1_b, fc1_gg, fc1_gb, gw1, gw2, g_b, g_gg, g_gb,
         fc2_wT, fc2_b, fc2_gg, fc2_gb, rel_pos, gmapC, gmapCT, gmap2C, gmap2CT):
    C = IN_CH
    row = lambda c: pl.BlockSpec((1, c), lambda b: (0, 0))
    full = lambda r, c: pl.BlockSpec((r, c), lambda b: (0, 0))
    return pl.pallas_call(
        _body,
        grid=(B,),
        in_specs=[
            pl.BlockSpec((1, N, C), lambda b: (b, 0, 0)),
            pl.BlockSpec(memory_space=pl.ANY), row(C), row(C), row(C),
            pl.BlockSpec(memory_space=pl.ANY),
            pl.BlockSpec(memory_space=pl.ANY),
            row(2 * C), row(2 * C), row(2 * C),
            pl.BlockSpec(memory_space=pl.ANY), row(C), row(C), row(C),
            pl.BlockSpec(memory_space=pl.ANY), full(C, GROUPS),
            full(GROUPS, C),
            full(2 * C, GROUPS), full(GROUPS, 2 * C),
        ],
        out_specs=pl.BlockSpec((1, N, C), lambda b: (b, 0, 0)),
        out_shape=jax.ShapeDtypeStruct((B, N, C), jnp.float32),
        scratch_shapes=[
            pltpu.VMEM((C, C), jnp.float32),
            pltpu.VMEM((C, 2 * C), jnp.bfloat16),
            pltpu.VMEM((C, 2 * C), jnp.bfloat16),
            pltpu.VMEM((2 * C, C), jnp.bfloat16),
            pltpu.VMEM((N, N), jnp.float32),
            pltpu.SemaphoreType.DMA,
            pltpu.SemaphoreType.DMA,
            pltpu.SemaphoreType.DMA,
            pltpu.SemaphoreType.DMA,
            pltpu.SemaphoreType.DMA,
        ],
        compiler_params=pltpu.CompilerParams(
            dimension_semantics=("arbitrary",),
        ),
    )(xf_in, fc1_wT, fc1_b, fc1_gg, fc1_gb, gw1, gw2, g_b, g_gg, g_gb,
      fc2_wT, fc2_b, fc2_gg, fc2_gb, rel_pos, gmapC, gmapCT, gmap2C, gmap2CT)


def kernel(x, fc1_w, fc1_b, fc1_gg, fc1_gb, g_w, g_b, g_gg, g_gb,
           fc2_w, fc2_b, fc2_gg, fc2_gb):
    Bx, C, Hx, Wx = x.shape
    xf_in = jnp.transpose(x.reshape(Bx, C, N), (0, 2, 1))
    rel_pos = jnp.asarray(_relative_pos_np(C, Hx))
    gmapC = jnp.asarray(_group_map(C, GROUPS))
    gmap2C = jnp.asarray(_group_map(2 * C, GROUPS))
    gw = jnp.transpose(g_w).astype(jnp.bfloat16)
    out = _run(
        xf_in, jnp.transpose(fc1_w), fc1_b[None, :], fc1_gg[None, :],
        fc1_gb[None, :], gw[:C], gw[C:], g_b[None, :], g_gg[None, :],
        g_gb[None, :], jnp.transpose(fc2_w).astype(jnp.bfloat16),
        fc2_b[None, :], fc2_gg[None, :],
        fc2_gb[None, :], rel_pos, gmapC, jnp.transpose(gmapC),
        gmap2C, jnp.transpose(gmap2C),
    )
    return jnp.transpose(out, (0, 2, 1)).reshape(Bx, C, Hx, Wx)


# speedup vs baseline: 1.0017x; 1.0017x over previous
import functools

import jax
import jax.numpy as jnp
import numpy as np
from jax.experimental import pallas as pl
from jax.experimental.pallas import tpu as pltpu

IN_CH = 768
K = 9
GROUPS = 32
H = W = 16
N = H * W
B = 16

_F32_MIN = -3.0e38


def _sincos_1d(embed_dim, pos):
    omega = np.arange(embed_dim // 2, dtype=np.float64)
    omega = omega / (embed_dim / 2.0)
    omega = 1.0 / (10000.0 ** omega)
    out = np.einsum('m,d->md', pos.reshape(-1).astype(np.float64), omega)
    return np.concatenate([np.sin(out), np.cos(out)], axis=1)


def _relative_pos_np(embed_dim, grid_size):
    gh = np.arange(grid_size, dtype=np.float32)
    gw = np.arange(grid_size, dtype=np.float32)
    grid = np.stack(np.meshgrid(gw, gh), axis=0)
    emb_h = _sincos_1d(embed_dim // 2, grid[0])
    emb_w = _sincos_1d(embed_dim // 2, grid[1])
    pe = np.concatenate([emb_h, emb_w], axis=1)
    rp = 2.0 * (pe @ pe.T) / pe.shape[1]
    return rp.astype(np.float32)


def _group_map(channels, groups):
    m = np.zeros((channels, groups), dtype=np.float32)
    per = channels // groups
    for g in range(groups):
        m[g * per:(g + 1) * per, g] = 1.0
    return m


def _dot(a, b, precision=jax.lax.Precision.HIGHEST):
    return jax.lax.dot_general(a, b, (((1,), (0,)), ((), ())),
                               preferred_element_type=jnp.float32,
                               precision=precision)


def _group_norm(y, gmap, gmapT, gg, gb, nelem):
    s = jnp.sum(y, axis=0, keepdims=True)
    ss = jnp.sum(y * y, axis=0, keepdims=True)
    gs = _dot(s, gmap)
    gss = _dot(ss, gmap)
    mu = gs / nelem
    var = gss / nelem - mu * mu
    rstd = jax.lax.rsqrt(var + 1e-5)
    mu_c = _dot(mu, gmapT)
    rstd_c = _dot(rstd, gmapT)
    a = rstd_c * gg
    b = gb - mu_c * a
    return y * a + b


def _erf(z):
    return jax.lax.erf(z)


def _body(x_ref, fc1_wT_ref, fc1_b_ref, fc1_gg_ref, fc1_gb_ref,
          gw1_ref, gw2_ref, g_b_ref, g_gg_ref, g_gb_ref,
          fc2_wT_ref, fc2_b_ref, fc2_gg_ref, fc2_gb_ref,
          rel_pos_ref, gmapC_ref, gmapCT_ref, gmap2C_ref, gmap2CT_ref,
          out_ref):
    x = x_ref[0]
    y = _dot(x, fc1_wT_ref[...], precision=None) + fc1_b_ref[...]
    xf = _group_norm(y, gmapC_ref[...], gmapCT_ref[...],
                     fc1_gg_ref[...], fc1_gb_ref[...],
                     float((IN_CH // GROUPS) * N))

    nrm = jnp.sqrt(jnp.sum(xf * xf, axis=1, keepdims=True))
    xn = xf / jnp.maximum(nrm, 1e-12)
    inner = jax.lax.dot_general(xn, xn, (((1,), (1,)), ((), ())),
                                preferred_element_type=jnp.float32,
                                precision=None)
    sq = jnp.sum(xn * xn, axis=1, keepdims=True)
    dist = 2.0 * inner - sq - jnp.transpose(sq) + rel_pos_ref[...]

    xf_bf = xf.astype(jnp.bfloat16)
    cols = jax.lax.broadcasted_iota(jnp.int32, (N, N), 1)
    d = dist
    acc = jnp.full((N, IN_CH), _F32_MIN, dtype=jnp.float32)
    for _ in range(K):
        m = jnp.max(d, axis=1, keepdims=True)
        amin = jnp.min(jnp.where(d == m, cols, N), axis=1, keepdims=True)
        onehot = (cols == amin).astype(jnp.bfloat16)
        acc = jnp.maximum(acc, _dot(onehot, xf_bf, precision=None))
        d = jnp.where(cols == amin, _F32_MIN, d)
    x_j = acc - xf

    t = (_dot(xf_bf, gw1_ref[...], precision=None)
         + _dot(x_j.astype(jnp.bfloat16), gw2_ref[...], precision=None)
         + g_b_ref[...])
    t = _group_norm(t, gmap2C_ref[...], gmap2CT_ref[...],
                    g_gg_ref[...], g_gb_ref[...],
                    float((2 * IN_CH // GROUPS) * N))
    u = 0.5 * t * (1.0 + _erf(t * np.float32(1.0 / np.sqrt(2.0))))

    z = _dot(u.astype(jnp.bfloat16), fc2_wT_ref[...], precision=None) + fc2_b_ref[...]
    z = _group_norm(z, gmapC_ref[...], gmapCT_ref[...],
                    fc2_gg_ref[...], fc2_gb_ref[...],
                    float((IN_CH // GROUPS) * N))
    out_ref[0] = z + x


@functools.partial(jax.jit, static_argnames=())
def _run(xf_in, fc1_wT, fc1_b, fc1_gg, fc1_gb, gw1, gw2, g_b, g_gg, g_gb,
         fc2_wT, fc2_b, fc2_gg, fc2_gb, rel_pos, gmapC, gmapCT, gmap2C, gmap2CT):
    C = IN_CH
    row = lambda c: pl.BlockSpec((1, c), lambda b: (0, 0))
    full = lambda r, c: pl.BlockSpec((r, c), lambda b: (0, 0))
    return pl.pallas_call(
        _body,
        grid=(B,),
        in_specs=[
            pl.BlockSpec((1, N, C), lambda b: (b, 0, 0)),
            full(C, C), row(C), row(C), row(C),
            full(C, 2 * C), full(C, 2 * C), row(2 * C), row(2 * C), row(2 * C),
            full(2 * C, C), row(C), row(C), row(C),
            full(N, N), full(C, GROUPS), full(GROUPS, C),
            full(2 * C, GROUPS), full(GROUPS, 2 * C),
        ],
        out_specs=pl.BlockSpec((1, N, C), lambda b: (b, 0, 0)),
        out_shape=jax.ShapeDtypeStruct((B, N, C), jnp.float32),
        compiler_params=pltpu.CompilerParams(
            dimension_semantics=("arbitrary",),
        ),
    )(xf_in, fc1_wT, fc1_b, fc1_gg, fc1_gb, gw1, gw2, g_b, g_gg, g_gb,
      fc2_wT, fc2_b, fc2_gg, fc2_gb, rel_pos, gmapC, gmapCT, gmap2C, gmap2CT)


def kernel(x, fc1_w, fc1_b, fc1_gg, fc1_gb, g_w, g_b, g_gg, g_gb,
           fc2_w, fc2_b, fc2_gg, fc2_gb):
    Bx, C, Hx, Wx = x.shape
    xf_in = jnp.transpose(x.reshape(Bx, C, N), (0, 2, 1))
    rel_pos = jnp.asarray(_relative_pos_np(C, Hx))
    gmapC = jnp.asarray(_group_map(C, GROUPS))
    gmap2C = jnp.asarray(_group_map(2 * C, GROUPS))
    gw = jnp.transpose(g_w).astype(jnp.bfloat16)
    out = _run(
        xf_in, jnp.transpose(fc1_w), fc1_b[None, :], fc1_gg[None, :],
        fc1_gb[None, :], gw[:C], gw[C:], g_b[None, :], g_gg[None, :],
        g_gb[None, :], jnp.transpose(fc2_w).astype(jnp.bfloat16),
        fc2_b[None, :], fc2_gg[None, :],
        fc2_gb[None, :], rel_pos, gmapC, jnp.transpose(gmapC),
        gmap2C, jnp.transpose(gmap2C),
    )
    return jnp.transpose(out, (0, 2, 1)).reshape(Bx, C, Hx, Wx)
